# Initial kernel scaffold; baseline (speedup 1.0000x reference)
import jax
import jax.numpy as jnp
from jax.experimental import pallas as pl


def kernel(x, task_embedding, W, b, g1w_w, g1w_b, g2w_w, g2w_b, g1b_w, g1b_b, g2b_w, g2b_b):
    raise NotImplementedError("write your pallas kernel here")


# speedup vs baseline: 1.2312x; 1.2312x over previous
import jax
import jax.numpy as jnp
from jax.experimental import pallas as pl

N = 8192
D = 256
TE = 128
HID_W = 2048
FLAT_W = D * D
K_W = int((1.0 - 0.9) * FLAT_W)
K_B = int((1.0 - 0.9) * D)

ROWS_B = 2048
GRID_B = FLAT_W // ROWS_B
KC = 128
ROWS_D = 1024
GRID_D = N // ROWS_D

_DN = (((1,), (1,)), ((), ()))


def _gen_small_kernel(te_ref, g1w_w_ref, g1w_b_ref, g1b_w_ref, g1b_b_ref,
                      g2b_w_ref, g2b_b_ref, hw_ref, sb_ref):
    te = te_ref[...]
    hw = jax.lax.dot_general(te, g1w_w_ref[...], _DN,
                             preferred_element_type=jnp.float32)
    hw_ref[...] = jax.nn.relu(hw + g1w_b_ref[...])
    hb = jax.nn.relu(jax.lax.dot_general(te, g1b_w_ref[...], _DN,
                                         preferred_element_type=jnp.float32)
                     + g1b_b_ref[...])
    zb = jax.lax.dot_general(hb, g2b_w_ref[...], _DN,
                             preferred_element_type=jnp.float32)
    sb_ref[...] = jax.nn.sigmoid(zb + g2b_b_ref[...])


def _scores_kernel(g2w_ref, hw_ref, g2wb_ref, out_ref):
    acc = jax.lax.dot_general(hw_ref[:, :KC], g2w_ref[:, :KC], _DN,
                              preferred_element_type=jnp.float32)
    for c in range(1, HID_W // KC):
        acc = acc + jax.lax.dot_general(
            hw_ref[:, c * KC:(c + 1) * KC], g2w_ref[:, c * KC:(c + 1) * KC],
            _DN, preferred_element_type=jnp.float32)
    out_ref[...] = jax.nn.sigmoid(acc + g2wb_ref[...])


def _kth_largest_bits(s, k):
    def body(i, cur):
        cand = cur | jnp.left_shift(jnp.int32(1), jnp.int32(30) - i)
        cnt = jnp.sum((s >= cand).astype(jnp.int32))
        return jnp.where(cnt >= k, cand, cur)
    return jax.lax.fori_loop(0, 31, body, jnp.int32(0))


def _mask_kernel(sw_ref, sb_ref, W_ref, b_ref, wm_ref, bm_ref):
    sw = jax.lax.bitcast_convert_type(sw_ref[...], jnp.int32)
    thr_w = _kth_largest_bits(sw, K_W)
    wm_ref[...] = W_ref[...] * (sw >= thr_w).astype(jnp.float32)
    sb = jax.lax.bitcast_convert_type(sb_ref[...], jnp.int32)
    thr_b = _kth_largest_bits(sb, K_B)
    bm_ref[...] = b_ref[...] * (sb >= thr_b).astype(jnp.float32)


def _fwd_kernel(x_ref, wm_ref, bm_ref, out_ref):
    out_ref[...] = jax.lax.dot_general(
        x_ref[...], wm_ref[...], _DN,
        preferred_element_type=jnp.float32) + bm_ref[...]


def kernel(x, task_embedding, W, b, g1w_w, g1w_b, g2w_w, g2w_b,
           g1b_w, g1b_b, g2b_w, g2b_b):
    te = task_embedding.reshape(1, TE)
    hw, sb = pl.pallas_call(
        _gen_small_kernel,
        out_shape=(jax.ShapeDtypeStruct((1, HID_W), jnp.float32),
                   jax.ShapeDtypeStruct((1, D), jnp.float32)),
    )(te, g1w_w, g1w_b.reshape(1, HID_W), g1b_w, g1b_b.reshape(1, TE),
      g2b_w, g2b_b.reshape(1, D))

    scores_w = pl.pallas_call(
        _scores_kernel,
        grid=(GRID_B,),
        in_specs=[
            pl.BlockSpec((ROWS_B, HID_W), lambda i: (i, 0)),
            pl.BlockSpec((1, HID_W), lambda i: (0, 0)),
            pl.BlockSpec((1, ROWS_B), lambda i: (0, i)),
        ],
        out_specs=pl.BlockSpec((1, ROWS_B), lambda i: (0, i)),
        out_shape=jax.ShapeDtypeStruct((1, FLAT_W), jnp.float32),
    )(g2w_w, hw, g2w_b.reshape(1, FLAT_W))

    wm, bm = pl.pallas_call(
        _mask_kernel,
        out_shape=(jax.ShapeDtypeStruct((FLAT_W // 128, 128), jnp.float32),
                   jax.ShapeDtypeStruct((1, D), jnp.float32)),
    )(scores_w.reshape(FLAT_W // 128, 128), sb, W.reshape(FLAT_W // 128, 128),
      b.reshape(1, D))
    wm = wm.reshape(D, D)

    out = pl.pallas_call(
        _fwd_kernel,
        grid=(GRID_D,),
        in_specs=[
            pl.BlockSpec((ROWS_D, D), lambda i: (i, 0)),
            pl.BlockSpec((D, D), lambda i: (0, 0)),
            pl.BlockSpec((1, D), lambda i: (0, 0)),
        ],
        out_specs=pl.BlockSpec((ROWS_D, D), lambda i: (i, 0)),
        out_shape=jax.ShapeDtypeStruct((N, D), jnp.float32),
    )(x, wm, bm)
    return out


# speedup vs baseline: 1.2563x; 1.0204x over previous
import jax
import jax.numpy as jnp
from jax.experimental import pallas as pl

N = 8192
D = 256
TE = 128
HID_W = 2048
FLAT_W = D * D
K_W = int((1.0 - 0.9) * FLAT_W)
K_B = int((1.0 - 0.9) * D)

ROWS_B = 1024
GRID_B = FLAT_W // ROWS_B
KC = 128
ROWS_D = 1024
GRID_D = N // ROWS_D

_DN = (((1,), (1,)), ((), ()))


def _gen_small_kernel(te_ref, g1w_w_ref, g1w_b_ref, g1b_w_ref, g1b_b_ref,
                      g2b_w_ref, g2b_b_ref, hw_ref, sb_ref):
    te = te_ref[...]
    hw = jax.lax.dot_general(te, g1w_w_ref[...], _DN,
                             preferred_element_type=jnp.float32)
    hw_ref[...] = jax.nn.relu(hw + g1w_b_ref[...])
    hb = jax.nn.relu(jax.lax.dot_general(te, g1b_w_ref[...], _DN,
                                         preferred_element_type=jnp.float32)
                     + g1b_b_ref[...])
    zb = jax.lax.dot_general(hb, g2b_w_ref[...], _DN,
                             preferred_element_type=jnp.float32)
    sb_ref[...] = jax.nn.sigmoid(zb + g2b_b_ref[...])


def _scores_kernel(g2w_ref, hw_ref, g2wb_ref, out_ref):
    acc = jax.lax.dot_general(hw_ref[:, :KC], g2w_ref[:, :KC], _DN,
                              preferred_element_type=jnp.float32)
    for c in range(1, HID_W // KC):
        acc = acc + jax.lax.dot_general(
            hw_ref[:, c * KC:(c + 1) * KC], g2w_ref[:, c * KC:(c + 1) * KC],
            _DN, preferred_element_type=jnp.float32)
    out_ref[...] = jax.nn.sigmoid(acc + g2wb_ref[...])


def _kth_largest_bits(s, k):
    def body(i, cur):
        cand = cur | jnp.left_shift(jnp.int32(1), jnp.int32(30) - i)
        cnt = jnp.sum((s >= cand).astype(jnp.int32))
        return jnp.where(cnt >= k, cand, cur)
    return jax.lax.fori_loop(0, 31, body, jnp.int32(0))


def _mask_kernel(sw_ref, sb_ref, W_ref, b_ref, wm_ref, bm_ref):
    sw = jax.lax.bitcast_convert_type(sw_ref[...], jnp.int32)
    thr_w = _kth_largest_bits(sw, K_W)
    wm_ref[...] = W_ref[...] * (sw >= thr_w).astype(jnp.float32)
    sb = jax.lax.bitcast_convert_type(sb_ref[...], jnp.int32)
    thr_b = _kth_largest_bits(sb, K_B)
    bm_ref[...] = b_ref[...] * (sb >= thr_b).astype(jnp.float32)


def _fwd_kernel(x_ref, wm_ref, bm_ref, out_ref):
    out_ref[...] = jax.lax.dot_general(
        x_ref[...], wm_ref[...], _DN,
        preferred_element_type=jnp.float32) + bm_ref[...]


def kernel(x, task_embedding, W, b, g1w_w, g1w_b, g2w_w, g2w_b,
           g1b_w, g1b_b, g2b_w, g2b_b):
    te = task_embedding.reshape(1, TE)
    hw, sb = pl.pallas_call(
        _gen_small_kernel,
        out_shape=(jax.ShapeDtypeStruct((1, HID_W), jnp.float32),
                   jax.ShapeDtypeStruct((1, D), jnp.float32)),
    )(te, g1w_w, g1w_b.reshape(1, HID_W), g1b_w, g1b_b.reshape(1, TE),
      g2b_w, g2b_b.reshape(1, D))

    scores_w = pl.pallas_call(
        _scores_kernel,
        grid=(GRID_B,),
        in_specs=[
            pl.BlockSpec((ROWS_B, HID_W), lambda i: (i, 0)),
            pl.BlockSpec((1, HID_W), lambda i: (0, 0)),
            pl.BlockSpec((1, ROWS_B), lambda i: (0, i)),
        ],
        out_specs=pl.BlockSpec((1, ROWS_B), lambda i: (0, i)),
        out_shape=jax.ShapeDtypeStruct((1, FLAT_W), jnp.float32),
    )(g2w_w, hw, g2w_b.reshape(1, FLAT_W))

    wm, bm = pl.pallas_call(
        _mask_kernel,
        out_shape=(jax.ShapeDtypeStruct((FLAT_W // 128, 128), jnp.float32),
                   jax.ShapeDtypeStruct((1, D), jnp.float32)),
    )(scores_w.reshape(FLAT_W // 128, 128), sb, W.reshape(FLAT_W // 128, 128),
      b.reshape(1, D))
    wm = wm.reshape(D, D)

    out = pl.pallas_call(
        _fwd_kernel,
        grid=(GRID_D,),
        in_specs=[
            pl.BlockSpec((ROWS_D, D), lambda i: (i, 0)),
            pl.BlockSpec((D, D), lambda i: (0, 0)),
            pl.BlockSpec((1, D), lambda i: (0, 0)),
        ],
        out_specs=pl.BlockSpec((ROWS_D, D), lambda i: (i, 0)),
        out_shape=jax.ShapeDtypeStruct((N, D), jnp.float32),
    )(x, wm, bm)
    return out
